# baseline (speedup 1.0000x reference)
import jax
import jax.numpy as jnp
from jax import lax
from jax.experimental import pallas as pl
from jax.experimental.pallas import tpu as pltpu
from jax.experimental.pallas import tpu_sc as plsc

NN = 10000
EE = 320000
FIN = 128
FOUT = 256
NCLS = 64

NC, NS = 2, 16
K = 80
NBUF = 3
SI = 4
ROWS_ALL = EE // K
ROWS_TILE_A = ROWS_ALL // NS
ROWS_TILE_C = ROWS_ALL // (NC * NS)
RPT = NN // NS

DS = 16
DF = 7
KF_D = 8


def _deg_body(src_hbm, dst_hbm, z1_hbm, deg_hbm, idx_v, ones_v, tb_v, acc_s,
              dsem, isem):
    c = lax.axis_index("c")
    s = lax.axis_index("s")

    @pl.when(s < 10)
    def _():
        pltpu.sync_copy(z1_hbm, tb_v)
        pltpu.sync_copy(tb_v, acc_s.at[pl.ds(s * 1000, 1000)])

    for u in range(K // 16):
        ones_v[pl.ds(u * 16, 16)] = jnp.full((16,), 1.0, jnp.float32)

    plsc.subcore_barrier()

    base = s * (EE // NS)

    def fire_i(j, p):
        @pl.when(c == 0)
        def _():
            pltpu.async_copy(src_hbm.at[pl.ds(base + j * K, K)], idx_v.at[p],
                             isem.at[p])

        @pl.when(c == 1)
        def _():
            pltpu.async_copy(dst_hbm.at[pl.ds(base + j * K, K)], idx_v.at[p],
                             isem.at[p])

    def wait_i(j, p):
        pltpu.make_async_copy(src_hbm.at[pl.ds(base + j * K, K)], idx_v.at[p],
                              isem.at[p]).wait()

    for j in range(DF):
        fire_i(j, j % DS)

    def chunk(j, carry):
        p = lax.rem(j, DS)
        wait_i(j, p)

        @pl.when(j + DF < ROWS_TILE_A)
        def _():
            fire_i(j + DF, lax.rem(j + DF, DS))

        pltpu.async_copy(ones_v, acc_s.at[idx_v.at[p]], dsem, add=True)

        @pl.when(j >= KF_D)
        def _():
            pltpu.make_async_copy(ones_v, acc_s.at[idx_v.at[p]], dsem).wait()

        return carry

    lax.fori_loop(0, ROWS_TILE_A, chunk, None)
    for j in range(KF_D):
        pltpu.make_async_copy(ones_v, acc_s.at[idx_v.at[0]], dsem).wait()
    plsc.subcore_barrier()

    @pl.when(s < 10)
    def _():
        pltpu.sync_copy(acc_s.at[pl.ds(s * 1000, 1000)], tb_v)
        pltpu.sync_copy(tb_v, deg_hbm.at[pl.ds(c * NN + s * 1000, 1000)])


_sc_calls = {}


def _get_sc_calls():
    if _sc_calls:
        return _sc_calls
    mesh = plsc.VectorSubcoreMesh(
        core_axis_name="c", subcore_axis_name="s",
        num_cores=NC, num_subcores=NS)
    _sc_calls["deg"] = pl.kernel(
        _deg_body,
        out_type=jax.ShapeDtypeStruct((2 * NN,), jnp.float32),
        mesh=mesh,
        scratch_types=[
            pltpu.VMEM((DS, K), jnp.int32),
            pltpu.VMEM((K,), jnp.float32),
            pltpu.VMEM((1000,), jnp.float32),
            pltpu.VMEM_SHARED((NN,), jnp.float32),
            pltpu.SemaphoreType.DMA,
            pltpu.SemaphoreType.DMA((DS,)),
        ],
    )
    _sc_calls["agg"] = pl.kernel(
        _agg_body,
        out_type=(jax.ShapeDtypeStruct((2 * NN, FIN), jnp.float32),
                  jax.ShapeDtypeStruct((2 * NN,), jnp.float32)),
        mesh=mesh,
        scratch_types=[
            pltpu.VMEM((SI, K), jnp.int32),
            pltpu.VMEM((SI, K), jnp.int32),
            pltpu.VMEM((NBUF, K, FIN), jnp.float32),
            pltpu.VMEM((NBUF, K), jnp.float32),
            pltpu.VMEM((BR, FIN), jnp.float32),
            pltpu.VMEM((1000,), jnp.float32),
            pltpu.VMEM_SHARED((NN, FIN), jnp.float32),
            pltpu.VMEM_SHARED((NN,), jnp.float32),
            pltpu.VMEM_SHARED((NN,), jnp.float32),
            pltpu.SemaphoreType.DMA((NBUF,)),
            pltpu.SemaphoreType.DMA((NBUF,)),
            pltpu.SemaphoreType.DMA((NBUF,)),
            pltpu.SemaphoreType.DMA((NBUF,)),
            pltpu.SemaphoreType.DMA((SI,)),
        ],
    )
    return _sc_calls


BR = 16
NBC = 39
TAIL = NN - NS * NBC * BR


def _agg_body(src_hbm, dst_hbm, hs_hbm, rin_hbm, z2_hbm, z1_hbm,
              agg_hbm, t_hbm, sidx_v, didx_v, rows_v, rv_v, zb_v, tb_v,
              acc_s, tacc_s, rin_s, gsem, grsem, ssem, stsem, isem):
    c = lax.axis_index("c")
    s = lax.axis_index("s")

    @pl.when(s < 10)
    def _():
        pltpu.sync_copy(z1_hbm, tb_v)
        pltpu.sync_copy(tb_v, tacc_s.at[pl.ds(s * 1000, 1000)])
        pltpu.sync_copy(rin_hbm.at[pl.ds(s * 1000, 1000)], tb_v)
        pltpu.sync_copy(tb_v, rin_s.at[pl.ds(s * 1000, 1000)])

    pltpu.sync_copy(z2_hbm, zb_v)
    base = s * NBC * BR
    for q in range(NBC):
        pltpu.sync_copy(zb_v, acc_s.at[pl.ds(base + q * BR, BR)])

    @pl.when(s == 0)
    def _():
        pltpu.sync_copy(zb_v.at[pl.ds(0, TAIL)], acc_s.at[pl.ds(NN - TAIL, TAIL)])

    tile_base = (c * NS + s) * (EE // (NC * NS))
    plsc.subcore_barrier()

    def fire_i(j, q):
        pltpu.async_copy(src_hbm.at[pl.ds(tile_base + j * K, K)],
                         sidx_v.at[q], isem.at[q])
        pltpu.async_copy(dst_hbm.at[pl.ds(tile_base + j * K, K)],
                         didx_v.at[q], isem.at[q])

    def wait_i(j, q):
        pltpu.make_async_copy(src_hbm.at[pl.ds(tile_base + j * K, K)],
                              sidx_v.at[q], isem.at[q]).wait()
        pltpu.make_async_copy(dst_hbm.at[pl.ds(tile_base + j * K, K)],
                              didx_v.at[q], isem.at[q]).wait()

    def fire_g(j, p, q):
        pltpu.async_copy(hs_hbm.at[sidx_v.at[q]], rows_v.at[p], gsem.at[p])
        pltpu.async_copy(rin_s.at[didx_v.at[q]], rv_v.at[p], grsem.at[p])

    def wait_g(j, p, q):
        pltpu.make_async_copy(hs_hbm.at[sidx_v.at[q]], rows_v.at[p],
                              gsem.at[p]).wait()
        pltpu.make_async_copy(rin_s.at[didx_v.at[q]], rv_v.at[p],
                              grsem.at[p]).wait()

    def fire_s(j, p, q):
        pltpu.async_copy(rows_v.at[p], acc_s.at[didx_v.at[q]], ssem.at[p],
                         add=True)
        pltpu.async_copy(rv_v.at[p], tacc_s.at[sidx_v.at[q]], stsem.at[p],
                         add=True)

    def wait_s(j, p, q):
        pltpu.make_async_copy(rows_v.at[p], acc_s.at[didx_v.at[q]],
                              ssem.at[p]).wait()
        pltpu.make_async_copy(rv_v.at[p], tacc_s.at[sidx_v.at[q]],
                              stsem.at[p]).wait()

    fire_i(0, 0)
    fire_i(1, 1)
    fire_i(2, 2)
    wait_i(0, 0)
    fire_g(0, 0, 0)
    wait_i(1, 1)
    fire_g(1, 1, 1)

    def chunk(j, carry):
        p = lax.rem(j, NBUF)
        q = lax.rem(j, SI)

        wait_g(j, p, q)

        @pl.when(j > 0)
        def _():
            wait_s(j - 1, lax.rem(j - 1, NBUF), lax.rem(j - 1, SI))

        @pl.when(j + 2 < ROWS_TILE_C)
        def _():
            wait_i(j + 2, lax.rem(j + 2, SI))
            fire_g(j + 2, lax.rem(j + 2, NBUF), lax.rem(j + 2, SI))

        @pl.when(j + 3 < ROWS_TILE_C)
        def _():
            fire_i(j + 3, lax.rem(j + 3, SI))

        fire_s(j, p, q)
        return carry

    lax.fori_loop(0, ROWS_TILE_C, chunk, None)
    wait_s(ROWS_TILE_C - 1, (ROWS_TILE_C - 1) % NBUF, (ROWS_TILE_C - 1) % SI)
    plsc.subcore_barrier()

    for q in range(NBC):
        pltpu.sync_copy(acc_s.at[pl.ds(base + q * BR, BR)], zb_v)
        pltpu.sync_copy(zb_v, agg_hbm.at[pl.ds(c * NN + base + q * BR, BR)])

    @pl.when(s == 0)
    def _():
        pltpu.sync_copy(acc_s.at[pl.ds(NN - TAIL, TAIL)],
                        zb_v.at[pl.ds(0, TAIL)])
        pltpu.sync_copy(zb_v.at[pl.ds(0, TAIL)],
                        agg_hbm.at[pl.ds(c * NN + NN - TAIL, TAIL)])

    @pl.when(s < 10)
    def _():
        pltpu.sync_copy(tacc_s.at[pl.ds(s * 1000, 1000)], tb_v)
        pltpu.sync_copy(tb_v, t_hbm.at[pl.ds(c * NN + s * 1000, 1000)])


def _scale_body(h_ref, deg_ref, hs_ref, rio_ref, rib_ref):
    rr = lax.rsqrt(jnp.maximum(deg_ref[...], 1.0))
    rt = jnp.transpose(rr)
    hs_ref[...] = h_ref[...] * rt[:, 0:1]
    rio_ref[...] = rr
    rib_ref[...] = jnp.broadcast_to(rt[:, 1:2], (NN, FIN))


_scale_call = pl.pallas_call(
    _scale_body,
    out_shape=[
        jax.ShapeDtypeStruct((NN, FIN), jnp.float32),
        jax.ShapeDtypeStruct((2, NN), jnp.float32),
        jax.ShapeDtypeStruct((NN, FIN), jnp.float32),
    ],
)


def _out_body(p_ref, t_ref, rio_ref, rib_ref, w1_ref, b1_ref, w2_ref, b2_ref,
              o_ref):
    aggp = p_ref[pl.ds(0, NN), :] + p_ref[pl.ds(NN, NN), :]
    x1 = jnp.dot(aggp * rib_ref[...], w1_ref[...],
                 preferred_element_type=jnp.float32)
    x1 = jnp.maximum(x1 + b1_ref[...], 0.0)
    w = rio_ref[0:1, :] * (t_ref[0:1, :] + t_ref[1:2, :])
    v = lax.dot_general(w, x1, (((1,), (0,)), ((), ())),
                        preferred_element_type=jnp.float32)
    o_ref[...] = (jnp.dot(v, w2_ref[...],
                          preferred_element_type=jnp.float32) * (1.0 / NN)
                  + b2_ref[...])


_out_call = pl.pallas_call(
    _out_body,
    out_shape=jax.ShapeDtypeStruct((1, NCLS), jnp.float32),
)


def kernel(h, edge_index, W1, b1, W2, b2):
    ei = edge_index.astype(jnp.int32)
    src2 = ei[0]
    dst2 = ei[1]
    z1 = jnp.zeros((1000,), jnp.float32)
    z2 = jnp.zeros((BR, FIN), jnp.float32)

    sc = _get_sc_calls()
    deg = sc["deg"](src2, dst2, z1)

    hs, rio, rib = _scale_call(h, deg.reshape(2, NN))

    aggp, tp = sc["agg"](src2, dst2, hs, rio[1], z2, z1)

    return _out_call(aggp, tp.reshape(2, NN), rio, rib,
                     W1, b1.reshape(1, FOUT), W2, b2.reshape(1, NCLS))


# speedup vs baseline: 1.0149x; 1.0149x over previous
import jax
import jax.numpy as jnp
from jax import lax
from jax.experimental import pallas as pl
from jax.experimental.pallas import tpu as pltpu
from jax.experimental.pallas import tpu_sc as plsc

NN = 10000
EE = 320000
FIN = 128
FOUT = 256
NCLS = 64

NC, NS = 2, 16
K = 80
NBUF = 3
ROWS_ALL = EE // K
ROWS_TILE_A = ROWS_ALL // NS
ROWS_TILE_C = ROWS_ALL // (NC * NS)
RPT = NN // NS

WIN = 5
NWIN = 25


def _deg_body(src3_hbm, dst3_hbm, z1_hbm, deg_hbm, idx_v, ones_v, tb_v, acc_s,
              dsem):
    c = lax.axis_index("c")
    s = lax.axis_index("s")

    @pl.when(s < 10)
    def _():
        pltpu.sync_copy(z1_hbm, tb_v)
        pltpu.sync_copy(tb_v, acc_s.at[pl.ds(s * 1000, 1000)])

    for u in range(K // 16):
        ones_v[pl.ds(u * 16, 16)] = jnp.full((16,), 1.0, jnp.float32)

    @pl.when(c == 0)
    def _():
        pltpu.sync_copy(src3_hbm.at[2 * s], idx_v.at[0])
        pltpu.sync_copy(src3_hbm.at[2 * s + 1], idx_v.at[1])

    @pl.when(c == 1)
    def _():
        pltpu.sync_copy(dst3_hbm.at[2 * s], idx_v.at[0])
        pltpu.sync_copy(dst3_hbm.at[2 * s + 1], idx_v.at[1])

    plsc.subcore_barrier()

    KF = 8

    def row(j):
        jj = lax.rem(j, NWIN * WIN)
        return (idx_v.at[lax.div(j, NWIN * WIN)]
                .at[lax.div(jj, WIN)].at[lax.rem(jj, WIN)])

    def chunk(j, carry):
        pltpu.async_copy(ones_v, acc_s.at[row(j)], dsem, add=True)

        @pl.when(j >= KF)
        def _():
            pltpu.make_async_copy(ones_v, acc_s.at[row(j)], dsem).wait()

        return carry

    lax.fori_loop(0, ROWS_TILE_A, chunk, None)
    for j in range(KF):
        pltpu.make_async_copy(ones_v, acc_s.at[row(j)], dsem).wait()
    plsc.subcore_barrier()

    @pl.when(s < 10)
    def _():
        pltpu.sync_copy(acc_s.at[pl.ds(s * 1000, 1000)], tb_v)
        pltpu.sync_copy(tb_v, deg_hbm.at[pl.ds(c * NN + s * 1000, 1000)])


_sc_calls = {}


def _get_sc_calls():
    if _sc_calls:
        return _sc_calls
    mesh = plsc.VectorSubcoreMesh(
        core_axis_name="c", subcore_axis_name="s",
        num_cores=NC, num_subcores=NS)
    _sc_calls["deg"] = pl.kernel(
        _deg_body,
        out_type=jax.ShapeDtypeStruct((2 * NN,), jnp.float32),
        mesh=mesh,
        scratch_types=[
            pltpu.VMEM((2, NWIN, WIN, K), jnp.int32),
            pltpu.VMEM((K,), jnp.float32),
            pltpu.VMEM((1000,), jnp.float32),
            pltpu.VMEM_SHARED((NN,), jnp.float32),
            pltpu.SemaphoreType.DMA,
        ],
    )
    _sc_calls["agg"] = pl.kernel(
        _agg_body,
        out_type=(jax.ShapeDtypeStruct((2 * NN, FIN), jnp.float32),
                  jax.ShapeDtypeStruct((2 * NN,), jnp.float32)),
        mesh=mesh,
        scratch_types=[
            pltpu.VMEM((2, WIN, K), jnp.int32),
            pltpu.VMEM((2, WIN, K), jnp.int32),
            pltpu.VMEM((NBUF, K, FIN), jnp.float32),
            pltpu.VMEM((NBUF, K), jnp.float32),
            pltpu.VMEM((BR, FIN), jnp.float32),
            pltpu.VMEM((1000,), jnp.float32),
            pltpu.VMEM_SHARED((NN, FIN), jnp.float32),
            pltpu.VMEM_SHARED((NN,), jnp.float32),
            pltpu.VMEM_SHARED((NN,), jnp.float32),
            pltpu.SemaphoreType.DMA((NBUF,)),
            pltpu.SemaphoreType.DMA((NBUF,)),
            pltpu.SemaphoreType.DMA((NBUF,)),
            pltpu.SemaphoreType.DMA((NBUF,)),
            pltpu.SemaphoreType.DMA,
        ],
    )
    return _sc_calls


BR = 16
NBC = 39
TAIL = NN - NS * NBC * BR


def _agg_body(src3_hbm, dst3_hbm, hs_hbm, rin_hbm, z2_hbm, z1_hbm,
              agg_hbm, t_hbm, sidx_v, didx_v, rows_v, rv_v, zb_v, tb_v,
              acc_s, tacc_s, rin_s, gsem, grsem, ssem, stsem, isem):
    c = lax.axis_index("c")
    s = lax.axis_index("s")

    @pl.when(s < 10)
    def _():
        pltpu.sync_copy(z1_hbm, tb_v)
        pltpu.sync_copy(tb_v, tacc_s.at[pl.ds(s * 1000, 1000)])
        pltpu.sync_copy(rin_hbm.at[pl.ds(s * 1000, 1000)], tb_v)
        pltpu.sync_copy(tb_v, rin_s.at[pl.ds(s * 1000, 1000)])

    pltpu.sync_copy(z2_hbm, zb_v)
    base = s * NBC * BR
    for q in range(NBC):
        pltpu.sync_copy(zb_v, acc_s.at[pl.ds(base + q * BR, BR)])

    @pl.when(s == 0)
    def _():
        pltpu.sync_copy(zb_v.at[pl.ds(0, TAIL)], acc_s.at[pl.ds(NN - TAIL, TAIL)])

    tile_w = c * NS + s

    def sidx(j):
        return sidx_v.at[lax.rem(lax.div(j, WIN), 2)].at[lax.rem(j, WIN)]

    def didx(j):
        return didx_v.at[lax.rem(lax.div(j, WIN), 2)].at[lax.rem(j, WIN)]

    pltpu.sync_copy(src3_hbm.at[tile_w].at[0], sidx_v.at[0])
    pltpu.sync_copy(dst3_hbm.at[tile_w].at[0], didx_v.at[0])
    plsc.subcore_barrier()

    def fire_g(j, p):
        pltpu.async_copy(hs_hbm.at[sidx(j)], rows_v.at[p], gsem.at[p])
        pltpu.async_copy(rin_s.at[didx(j)], rv_v.at[p], grsem.at[p])

    def wait_g(j, p):
        pltpu.make_async_copy(hs_hbm.at[sidx(j)], rows_v.at[p],
                              gsem.at[p]).wait()
        pltpu.make_async_copy(rin_s.at[didx(j)], rv_v.at[p],
                              grsem.at[p]).wait()

    def fire_s(j, p):
        pltpu.async_copy(rows_v.at[p], acc_s.at[didx(j)], ssem.at[p], add=True)
        pltpu.async_copy(rv_v.at[p], tacc_s.at[sidx(j)], stsem.at[p], add=True)

    def wait_s(j, p):
        pltpu.make_async_copy(rows_v.at[p], acc_s.at[didx(j)],
                              ssem.at[p]).wait()
        pltpu.make_async_copy(rv_v.at[p], tacc_s.at[sidx(j)],
                              stsem.at[p]).wait()

    def fire_idx(w, wp):
        pltpu.async_copy(src3_hbm.at[tile_w].at[w], sidx_v.at[wp], isem)
        pltpu.async_copy(dst3_hbm.at[tile_w].at[w], didx_v.at[wp], isem)

    def wait_idx(w, wp):
        pltpu.make_async_copy(src3_hbm.at[tile_w].at[w], sidx_v.at[wp],
                              isem).wait()
        pltpu.make_async_copy(dst3_hbm.at[tile_w].at[w], didx_v.at[wp],
                              isem).wait()

    fire_g(0, 0)
    fire_g(1, 1)

    def chunk(j, carry):
        p = lax.rem(j, NBUF)
        jw = lax.rem(j, WIN)
        w = lax.div(j, WIN)

        wait_g(j, p)

        @pl.when(j > 0)
        def _():
            wait_s(j - 1, lax.rem(j - 1, NBUF))

        @pl.when((jw == 0) & (w < NWIN - 1))
        def _():
            fire_idx(w + 1, lax.rem(w + 1, 2))

        @pl.when((jw == WIN - 3) & (w < NWIN - 1))
        def _():
            wait_idx(w + 1, lax.rem(w + 1, 2))

        @pl.when(j < ROWS_TILE_C - 2)
        def _():
            fire_g(j + 2, lax.rem(j + 2, NBUF))

        fire_s(j, p)
        return carry

    lax.fori_loop(0, ROWS_TILE_C, chunk, None)
    wait_s(ROWS_TILE_C - 1, (ROWS_TILE_C - 1) % NBUF)
    plsc.subcore_barrier()

    for q in range(NBC):
        pltpu.sync_copy(acc_s.at[pl.ds(base + q * BR, BR)], zb_v)
        pltpu.sync_copy(zb_v, agg_hbm.at[pl.ds(c * NN + base + q * BR, BR)])

    @pl.when(s == 0)
    def _():
        pltpu.sync_copy(acc_s.at[pl.ds(NN - TAIL, TAIL)],
                        zb_v.at[pl.ds(0, TAIL)])
        pltpu.sync_copy(zb_v.at[pl.ds(0, TAIL)],
                        agg_hbm.at[pl.ds(c * NN + NN - TAIL, TAIL)])

    @pl.when(s < 10)
    def _():
        pltpu.sync_copy(tacc_s.at[pl.ds(s * 1000, 1000)], tb_v)
        pltpu.sync_copy(tb_v, t_hbm.at[pl.ds(c * NN + s * 1000, 1000)])


def _scale_body(h_ref, deg_ref, hs_ref, rio_ref, rib_ref):
    rr = lax.rsqrt(jnp.maximum(deg_ref[...], 1.0))
    rt = jnp.transpose(rr)
    hs_ref[...] = h_ref[...] * rt[:, 0:1]
    rio_ref[...] = rr
    rib_ref[...] = jnp.broadcast_to(rt[:, 1:2], (NN, FIN))


_scale_call = pl.pallas_call(
    _scale_body,
    out_shape=[
        jax.ShapeDtypeStruct((NN, FIN), jnp.float32),
        jax.ShapeDtypeStruct((2, NN), jnp.float32),
        jax.ShapeDtypeStruct((NN, FIN), jnp.float32),
    ],
)


def _out_body(p_ref, t_ref, rio_ref, rib_ref, w1_ref, b1_ref, w2_ref, b2_ref,
              o_ref):
    aggp = p_ref[pl.ds(0, NN), :] + p_ref[pl.ds(NN, NN), :]
    x1 = jnp.dot(aggp * rib_ref[...], w1_ref[...],
                 preferred_element_type=jnp.float32)
    x1 = jnp.maximum(x1 + b1_ref[...], 0.0)
    w = rio_ref[0:1, :] * (t_ref[0:1, :] + t_ref[1:2, :])
    v = lax.dot_general(w, x1, (((1,), (0,)), ((), ())),
                        preferred_element_type=jnp.float32)
    o_ref[...] = (jnp.dot(v, w2_ref[...],
                          preferred_element_type=jnp.float32) * (1.0 / NN)
                  + b2_ref[...])


_out_call = pl.pallas_call(
    _out_body,
    out_shape=jax.ShapeDtypeStruct((1, NCLS), jnp.float32),
)


def kernel(h, edge_index, W1, b1, W2, b2):
    ei = edge_index.astype(jnp.int32)
    src2 = ei[0].reshape(NC * NS, NWIN, WIN, K)
    dst2 = ei[1].reshape(NC * NS, NWIN, WIN, K)
    z1 = jnp.zeros((1000,), jnp.float32)
    z2 = jnp.zeros((BR, FIN), jnp.float32)

    sc = _get_sc_calls()
    deg = sc["deg"](src2, dst2, z1)

    hs, rio, rib = _scale_call(h, deg.reshape(2, NN))

    aggp, tp = sc["agg"](src2, dst2, hs, rio[1], z2, z1)

    return _out_call(aggp, tp.reshape(2, NN), rio, rib,
                     W1, b1.reshape(1, FOUT), W2, b2.reshape(1, NCLS))
